# baseline (speedup 1.0000x reference)
import functools

import jax
import jax.numpy as jnp
from jax import lax
from jax.experimental import pallas as pl
from jax.experimental.pallas import tpu as pltpu
from jax.experimental.pallas import tpu_sc as plsc

N = 100000
D = 128
E = 100000
MSG = 16
TD = 16

BR = 2000

NC = 2
NS = 16
NW = NC * NS
E_PAD = 102400

_SC_MESH = plsc.VectorSubcoreMesh(core_axis_name="c", subcore_axis_name="s")


def _gather_rows(table, idx, d, kb=320):
    m = idx.shape[0]
    pw = m // NW
    nb = pw // kb
    assert m % (NW * kb) == 0 and nb % 2 == 0

    @functools.partial(
        pl.kernel,
        out_type=jax.ShapeDtypeStruct((m, d), jnp.float32),
        mesh=_SC_MESH,
        scratch_types=[
            pltpu.VMEM((pw,), jnp.int32),
            pltpu.VMEM((kb, d), jnp.float32),
            pltpu.VMEM((kb, d), jnp.float32),
            pltpu.SemaphoreType.DMA,
            pltpu.SemaphoreType.DMA,
        ],
    )
    def gk(idx_hbm, tbl_hbm, out_hbm, idx_v, buf0, buf1, sem0, sem1):
        wid = lax.axis_index("s") * NC + lax.axis_index("c")
        base = wid * pw
        pltpu.sync_copy(idx_hbm.at[pl.ds(base, pw)], idx_v)
        pltpu.async_copy(tbl_hbm.at[idx_v.at[pl.ds(0, kb)]], buf0, sem0)

        @pl.loop(0, nb, step=2)
        def _(b):
            pltpu.make_async_copy(tbl_hbm.at[pl.ds(0, kb)], buf0, sem0).wait()
            pltpu.async_copy(tbl_hbm.at[idx_v.at[pl.ds((b + 1) * kb, kb)]],
                             buf1, sem1)
            pltpu.sync_copy(buf0, out_hbm.at[pl.ds(base + b * kb, kb)])

            @pl.when(b + 2 < nb)
            def _():
                pltpu.async_copy(tbl_hbm.at[idx_v.at[pl.ds((b + 2) * kb, kb)]],
                                 buf0, sem0)

            pltpu.make_async_copy(tbl_hbm.at[pl.ds(0, kb)], buf1, sem1).wait()
            pltpu.sync_copy(buf1, out_hbm.at[pl.ds(base + (b + 1) * kb, kb)])

    return gk(idx, table)


def _dense_body(mem_ref, s_ref, wcat_ref, wnbr_ref, b_ref, h_ref, a_ref, b2_ref):
    mem = mem_ref[...]
    c3 = jnp.dot(mem, wcat_ref[...], preferred_element_type=jnp.float32)
    hpre = c3[:, :D] + jnp.dot(s_ref[...], wnbr_ref[...],
                               preferred_element_type=jnp.float32) + b_ref[...]
    h_ref[...] = jnp.tanh(hpre)
    a_ref[...] = c3[:, D:2 * D]
    b2_ref[...] = c3[:, 2 * D:3 * D]


def _dense_phase(memory, S, Wcat, W_nbr, b):
    grid = (N // BR,)
    return pl.pallas_call(
        _dense_body,
        grid=grid,
        in_specs=[
            pl.BlockSpec((BR, D), lambda i: (i, 0)),
            pl.BlockSpec((BR, D), lambda i: (i, 0)),
            pl.BlockSpec((D, 3 * D), lambda i: (0, 0)),
            pl.BlockSpec((D, D), lambda i: (0, 0)),
            pl.BlockSpec((1, D), lambda i: (0, 0)),
        ],
        out_specs=[
            pl.BlockSpec((BR, D), lambda i: (i, 0)),
            pl.BlockSpec((BR, D), lambda i: (i, 0)),
            pl.BlockSpec((BR, D), lambda i: (i, 0)),
        ],
        out_shape=[jax.ShapeDtypeStruct((N, D), jnp.float32)] * 3,
    )(memory, S, Wcat, W_nbr, b)


def _final_body(a_ref, b2g_ref, msgg_ref, dtw_ref, valid_ref, mem_ref,
                wu3_ref, wu4_ref, bupd_ref, wt_ref, out_ref):
    te = jnp.cos(dtw_ref[...] * wt_ref[...])
    pre = (a_ref[...] + b2g_ref[...]
           + jnp.dot(msgg_ref[...], wu3_ref[...], preferred_element_type=jnp.float32)
           + jnp.dot(te, wu4_ref[...], preferred_element_type=jnp.float32)
           + bupd_ref[...])
    m = jnp.tanh(pre)
    out_ref[...] = jnp.where(valid_ref[...] > 0, m, mem_ref[...])


def _final_phase(A, B2g, msg_g, dtw, valid, memory, Wu3, Wu4, b_upd, w_time):
    grid = (N // BR,)
    return pl.pallas_call(
        _final_body,
        grid=grid,
        in_specs=[
            pl.BlockSpec((BR, D), lambda i: (i, 0)),
            pl.BlockSpec((BR, D), lambda i: (i, 0)),
            pl.BlockSpec((BR, MSG), lambda i: (i, 0)),
            pl.BlockSpec((BR, 1), lambda i: (i, 0)),
            pl.BlockSpec((BR, 1), lambda i: (i, 0)),
            pl.BlockSpec((BR, D), lambda i: (i, 0)),
            pl.BlockSpec((MSG, D), lambda i: (0, 0)),
            pl.BlockSpec((TD, D), lambda i: (0, 0)),
            pl.BlockSpec((1, D), lambda i: (0, 0)),
            pl.BlockSpec((1, TD), lambda i: (0, 0)),
        ],
        out_specs=pl.BlockSpec((BR, D), lambda i: (i, 0)),
        out_shape=jax.ShapeDtypeStruct((N, D), jnp.float32),
    )(A, B2g, msg_g, dtw, valid, memory, Wu3, Wu4, b_upd, w_time)


def kernel(edge_index, t, msg, memory, last_update, W_self, W_nbr, b, W_upd, b_upd, w_time):
    src, dst = edge_index[0], edge_index[1]
    Wcat = jnp.concatenate([W_self, W_upd[:D], W_upd[D:2 * D]], axis=1)
    Wu3 = W_upd[2 * D:2 * D + MSG]
    Wu4 = W_upd[2 * D + MSG:]

    S = jnp.zeros((N, D), jnp.float32).at[dst].add(memory[src])

    H, A, B2 = _dense_phase(memory, S, Wcat, W_nbr, b[None, :])

    pad_ids = (jnp.arange(E_PAD - E, dtype=jnp.int32) * 7919) % N
    src_p = jnp.concatenate([src, pad_ids])
    dst_p = jnp.concatenate([dst, pad_ids])
    idx_all = jnp.concatenate([src_p, dst_p])
    hsd = _gather_rows(H, idx_all, D)
    h_src = hsd[:E]
    h_dst = hsd[E_PAD:E_PAD + E]

    p = jnp.full((N,), -1, jnp.int32)
    p = p.at[src].max(jnp.arange(E, dtype=jnp.int32))
    p = p.at[dst].max(jnp.arange(E, dtype=jnp.int32) + E)

    valid = (p >= 0).astype(jnp.int32)
    ew = jnp.where(p >= E, p - E, p)
    ew_c = jnp.clip(ew, 0, E - 1)
    opp = jnp.where(p >= E, src[ew_c], dst[ew_c])
    dtw = t[ew_c] - last_update
    new_lu = jnp.where(valid > 0, t[ew_c], last_update)
    msg_g = msg[ew_c]
    B2g = B2[opp]

    new_memory = _final_phase(A, B2g, msg_g, dtw[:, None], valid[:, None],
                              memory, Wu3, Wu4, b_upd[None, :], w_time[None, :])

    return (h_src, h_dst, new_memory, new_lu)


# speedup vs baseline: 1.8248x; 1.8248x over previous
import functools

import jax
import jax.numpy as jnp
from jax import lax
from jax.experimental import pallas as pl
from jax.experimental.pallas import tpu as pltpu
from jax.experimental.pallas import tpu_sc as plsc

N = 100000
D = 128
E = 100000
MSG = 16
TD = 16

BR = 2000

NC = 2
NS = 16
NW = NC * NS
E_PAD = 102400

_SC_MESH = plsc.VectorSubcoreMesh(core_axis_name="c", subcore_axis_name="s")


def _gather_rows(table, idx, d, kb=320):
    m = idx.shape[0]
    pw = m // NW
    nb = pw // kb
    assert m % (NW * kb) == 0 and nb % 2 == 0

    @functools.partial(
        pl.kernel,
        out_type=jax.ShapeDtypeStruct((m, d), jnp.float32),
        mesh=_SC_MESH,
        scratch_types=[
            pltpu.VMEM((pw,), jnp.int32),
            pltpu.VMEM((kb, d), jnp.float32),
            pltpu.VMEM((kb, d), jnp.float32),
            pltpu.SemaphoreType.DMA,
            pltpu.SemaphoreType.DMA,
        ],
    )
    def gk(idx_hbm, tbl_hbm, out_hbm, idx_v, buf0, buf1, sem0, sem1):
        wid = lax.axis_index("s") * NC + lax.axis_index("c")
        base = wid * pw
        pltpu.sync_copy(idx_hbm.at[pl.ds(base, pw)], idx_v)
        pltpu.async_copy(tbl_hbm.at[idx_v.at[pl.ds(0, kb)]], buf0, sem0)

        @pl.loop(0, nb, step=2)
        def _(b):
            pltpu.make_async_copy(tbl_hbm.at[pl.ds(0, kb)], buf0, sem0).wait()
            pltpu.async_copy(tbl_hbm.at[idx_v.at[pl.ds((b + 1) * kb, kb)]],
                             buf1, sem1)
            pltpu.sync_copy(buf0, out_hbm.at[pl.ds(base + b * kb, kb)])

            @pl.when(b + 2 < nb)
            def _():
                pltpu.async_copy(tbl_hbm.at[idx_v.at[pl.ds((b + 2) * kb, kb)]],
                                 buf0, sem0)

            pltpu.make_async_copy(tbl_hbm.at[pl.ds(0, kb)], buf1, sem1).wait()
            pltpu.sync_copy(buf1, out_hbm.at[pl.ds(base + (b + 1) * kb, kb)])

    return gk(idx, table)


def _dense_body(mem_ref, s_ref, wcat_ref, wnbr_ref, b_ref, h_ref, a_ref, b2_ref):
    mem = mem_ref[...]
    c3 = jnp.dot(mem, wcat_ref[...], preferred_element_type=jnp.float32)
    hpre = c3[:, :D] + jnp.dot(s_ref[...], wnbr_ref[...],
                               preferred_element_type=jnp.float32) + b_ref[...]
    h_ref[...] = jnp.tanh(hpre)
    a_ref[...] = c3[:, D:2 * D]
    b2_ref[...] = c3[:, 2 * D:3 * D]


def _dense_phase(memory, S, Wcat, W_nbr, b):
    grid = (N // BR,)
    return pl.pallas_call(
        _dense_body,
        grid=grid,
        in_specs=[
            pl.BlockSpec((BR, D), lambda i: (i, 0)),
            pl.BlockSpec((BR, D), lambda i: (i, 0)),
            pl.BlockSpec((D, 3 * D), lambda i: (0, 0)),
            pl.BlockSpec((D, D), lambda i: (0, 0)),
            pl.BlockSpec((1, D), lambda i: (0, 0)),
        ],
        out_specs=[
            pl.BlockSpec((BR, D), lambda i: (i, 0)),
            pl.BlockSpec((BR, D), lambda i: (i, 0)),
            pl.BlockSpec((BR, D), lambda i: (i, 0)),
        ],
        out_shape=[jax.ShapeDtypeStruct((N, D), jnp.float32)] * 3,
    )(memory, S, Wcat, W_nbr, b)


def _final_body(a_ref, b2g_ref, msgg_ref, dtw_ref, valid_ref, mem_ref,
                wu3_ref, wu4_ref, bupd_ref, wt_ref, out_ref):
    te = jnp.cos(dtw_ref[...] * wt_ref[...])
    pre = (a_ref[...] + b2g_ref[...]
           + jnp.dot(msgg_ref[...], wu3_ref[...], preferred_element_type=jnp.float32)
           + jnp.dot(te, wu4_ref[...], preferred_element_type=jnp.float32)
           + bupd_ref[...])
    m = jnp.tanh(pre)
    out_ref[...] = jnp.where(valid_ref[...] > 0, m, mem_ref[...])


def _final_phase(A, B2g, msg_g, dtw, valid, memory, Wu3, Wu4, b_upd, w_time):
    grid = (N // BR,)
    return pl.pallas_call(
        _final_body,
        grid=grid,
        in_specs=[
            pl.BlockSpec((BR, D), lambda i: (i, 0)),
            pl.BlockSpec((BR, D), lambda i: (i, 0)),
            pl.BlockSpec((BR, MSG), lambda i: (i, 0)),
            pl.BlockSpec((BR, 1), lambda i: (i, 0)),
            pl.BlockSpec((BR, 1), lambda i: (i, 0)),
            pl.BlockSpec((BR, D), lambda i: (i, 0)),
            pl.BlockSpec((MSG, D), lambda i: (0, 0)),
            pl.BlockSpec((TD, D), lambda i: (0, 0)),
            pl.BlockSpec((1, D), lambda i: (0, 0)),
            pl.BlockSpec((1, TD), lambda i: (0, 0)),
        ],
        out_specs=pl.BlockSpec((BR, D), lambda i: (i, 0)),
        out_shape=jax.ShapeDtypeStruct((N, D), jnp.float32),
    )(A, B2g, msg_g, dtw, valid, memory, Wu3, Wu4, b_upd, w_time)


def kernel(edge_index, t, msg, memory, last_update, W_self, W_nbr, b, W_upd, b_upd, w_time):
    src, dst = edge_index[0], edge_index[1]
    Wcat = jnp.concatenate([W_self, W_upd[:D], W_upd[D:2 * D]], axis=1)
    Wu3 = W_upd[2 * D:2 * D + MSG]
    Wu4 = W_upd[2 * D + MSG:]

    S = jnp.zeros((N, D), jnp.float32).at[dst].add(memory[src])

    H, A, B2 = _dense_phase(memory, S, Wcat, W_nbr, b[None, :])

    pad_ids = (jnp.arange(E_PAD - E, dtype=jnp.int32) * 7919) % N
    src_p = jnp.concatenate([src, pad_ids])
    dst_p = jnp.concatenate([dst, pad_ids])
    idx_all = jnp.concatenate([src_p, dst_p])
    hsd = _gather_rows(H, idx_all, D)
    h_src = hsd[:E]
    h_dst = hsd[E_PAD:E_PAD + E]

    p = jnp.full((N,), -1, jnp.int32)
    p = p.at[src].max(jnp.arange(E, dtype=jnp.int32))
    p = p.at[dst].max(jnp.arange(E, dtype=jnp.int32) + E)

    valid = (p >= 0).astype(jnp.int32)
    ew = jnp.where(p >= E, p - E, p)
    spread = jnp.arange(N, dtype=jnp.int32) % E
    ew_c = jnp.where(p >= 0, jnp.clip(ew, 0, E - 1), spread)
    opp = jnp.where(p >= E, src[ew_c], dst[ew_c])
    dtw = t[ew_c] - last_update
    new_lu = jnp.where(valid > 0, t[ew_c], last_update)
    msg_g = msg[ew_c]
    B2g = B2[opp]

    new_memory = _final_phase(A, B2g, msg_g, dtw[:, None], valid[:, None],
                              memory, Wu3, Wu4, b_upd[None, :], w_time[None, :])

    return (h_src, h_dst, new_memory, new_lu)


# speedup vs baseline: 2.7454x; 1.5045x over previous
import functools

import jax
import jax.numpy as jnp
from jax import lax
from jax.experimental import pallas as pl
from jax.experimental.pallas import tpu as pltpu
from jax.experimental.pallas import tpu_sc as plsc

N = 100000
D = 128
E = 100000
MSG = 16
TD = 16

BR = 2000

NC = 2
NS = 16
NW = NC * NS
E_PAD = 102400

_SC_MESH = plsc.VectorSubcoreMesh(core_axis_name="c", subcore_axis_name="s")


def _gather_rows(table, idx, d, kb=320):
    m = idx.shape[0]
    pw = m // NW
    nb = pw // kb
    assert m % (NW * kb) == 0 and nb % 2 == 0

    @functools.partial(
        pl.kernel,
        out_type=jax.ShapeDtypeStruct((m, d), jnp.float32),
        mesh=_SC_MESH,
        scratch_types=[
            pltpu.VMEM((pw,), jnp.int32),
            pltpu.VMEM((kb, d), jnp.float32),
            pltpu.VMEM((kb, d), jnp.float32),
            pltpu.SemaphoreType.DMA,
            pltpu.SemaphoreType.DMA,
        ],
    )
    def gk(idx_hbm, tbl_hbm, out_hbm, idx_v, buf0, buf1, sem0, sem1):
        wid = lax.axis_index("s") * NC + lax.axis_index("c")
        base = wid * pw
        pltpu.sync_copy(idx_hbm.at[pl.ds(base, pw)], idx_v)
        pltpu.async_copy(tbl_hbm.at[idx_v.at[pl.ds(0, kb)]], buf0, sem0)

        @pl.loop(0, nb, step=2)
        def _(b):
            pltpu.make_async_copy(tbl_hbm.at[pl.ds(0, kb)], buf0, sem0).wait()
            pltpu.async_copy(tbl_hbm.at[idx_v.at[pl.ds((b + 1) * kb, kb)]],
                             buf1, sem1)
            pltpu.sync_copy(buf0, out_hbm.at[pl.ds(base + b * kb, kb)])

            @pl.when(b + 2 < nb)
            def _():
                pltpu.async_copy(tbl_hbm.at[idx_v.at[pl.ds((b + 2) * kb, kb)]],
                                 buf0, sem0)

            pltpu.make_async_copy(tbl_hbm.at[pl.ds(0, kb)], buf1, sem1).wait()
            pltpu.sync_copy(buf1, out_hbm.at[pl.ds(base + (b + 1) * kb, kb)])

    return gk(idx, table)


N_PAD = 102400


def _winner(src_p, dst_p):
    epw = E_PAD // NW
    ngrp = epw // 16
    CHK = 10240
    NR = N_PAD // CHK
    mpt = CHK // NS

    @functools.partial(
        pl.kernel,
        out_type=jax.ShapeDtypeStruct((2, N_PAD), jnp.int32),
        mesh=_SC_MESH,
        compiler_params=pltpu.CompilerParams(needs_layout_passes=False),
        scratch_types=[
            pltpu.VMEM((epw,), jnp.int32),
            pltpu.VMEM((epw,), jnp.int32),
            pltpu.VMEM((N_PAD,), jnp.int32),
            pltpu.VMEM((NS, mpt), jnp.int32),
            pltpu.VMEM((mpt,), jnp.int32),
            pltpu.VMEM_SHARED((NS, CHK), jnp.int32),
        ],
    )
    def wk(src_hbm, dst_hbm, out_hbm, sbuf, dbuf, win, mbuf, tmp, shared):
        cid = lax.axis_index("c")
        sid = lax.axis_index("s")
        wid = sid * NC + cid
        ebase = wid * epw
        pltpu.sync_copy(src_hbm.at[pl.ds(ebase, epw)], sbuf)
        pltpu.sync_copy(dst_hbm.at[pl.ds(ebase, epw)], dbuf)

        @pl.loop(0, N_PAD // 16)
        def _(i):
            win[pl.ds(i * 16, 16)] = jnp.zeros((16,), jnp.int32)

        lanes = lax.iota(jnp.int32, 16)

        @pl.loop(0, ngrp)
        def _(g):
            pri0 = ebase + g * 16 + lanes + 1
            for buf, off in ((sbuf, 0), (dbuf, E_PAD)):
                ids = buf[pl.ds(g * 16, 16)]
                pri = pri0 + off

                def attempt(_):
                    cur = plsc.load_gather(win, [ids])
                    plsc.store_scatter(win, [ids], jnp.maximum(cur, pri))
                    cur2 = plsc.load_gather(win, [ids])
                    return jnp.any(cur2 < pri)

                lax.while_loop(lambda need: need, attempt, attempt(None))

        for k in range(NR):
            pltpu.sync_copy(win.at[pl.ds(k * CHK, CHK)], shared.at[sid])
            plsc.subcore_barrier()
            pltpu.sync_copy(shared.at[:, pl.ds(sid * mpt, mpt)], mbuf)

            @pl.loop(0, mpt // 16)
            def _(g):
                sl = pl.ds(g * 16, 16)
                acc = mbuf[0, sl]
                for r in range(1, NS):
                    acc = jnp.maximum(acc, mbuf[r, sl])
                tmp[sl] = acc

            pltpu.sync_copy(tmp, out_hbm.at[cid, pl.ds(k * CHK + sid * mpt, mpt)])
            plsc.subcore_barrier()

    return wk(src_p, dst_p)


def _dense_body(mem_ref, s_ref, wcat_ref, wnbr_ref, b_ref, h_ref, a_ref, b2_ref):
    mem = mem_ref[...]
    c3 = jnp.dot(mem, wcat_ref[...], preferred_element_type=jnp.float32)
    hpre = c3[:, :D] + jnp.dot(s_ref[...], wnbr_ref[...],
                               preferred_element_type=jnp.float32) + b_ref[...]
    h_ref[...] = jnp.tanh(hpre)
    a_ref[...] = c3[:, D:2 * D]
    b2_ref[...] = c3[:, 2 * D:3 * D]


def _dense_phase(memory, S, Wcat, W_nbr, b):
    grid = (N // BR,)
    return pl.pallas_call(
        _dense_body,
        grid=grid,
        in_specs=[
            pl.BlockSpec((BR, D), lambda i: (i, 0)),
            pl.BlockSpec((BR, D), lambda i: (i, 0)),
            pl.BlockSpec((D, 3 * D), lambda i: (0, 0)),
            pl.BlockSpec((D, D), lambda i: (0, 0)),
            pl.BlockSpec((1, D), lambda i: (0, 0)),
        ],
        out_specs=[
            pl.BlockSpec((BR, D), lambda i: (i, 0)),
            pl.BlockSpec((BR, D), lambda i: (i, 0)),
            pl.BlockSpec((BR, D), lambda i: (i, 0)),
        ],
        out_shape=[jax.ShapeDtypeStruct((N, D), jnp.float32)] * 3,
    )(memory, S, Wcat, W_nbr, b)


def _final_body(a_ref, b2g_ref, msgg_ref, dtw_ref, valid_ref, mem_ref,
                wu3_ref, wu4_ref, bupd_ref, wt_ref, out_ref):
    te = jnp.cos(dtw_ref[...] * wt_ref[...])
    pre = (a_ref[...] + b2g_ref[...]
           + jnp.dot(msgg_ref[...], wu3_ref[...], preferred_element_type=jnp.float32)
           + jnp.dot(te, wu4_ref[...], preferred_element_type=jnp.float32)
           + bupd_ref[...])
    m = jnp.tanh(pre)
    out_ref[...] = jnp.where(valid_ref[...] > 0, m, mem_ref[...])


def _final_phase(A, B2g, msg_g, dtw, valid, memory, Wu3, Wu4, b_upd, w_time):
    grid = (N // BR,)
    return pl.pallas_call(
        _final_body,
        grid=grid,
        in_specs=[
            pl.BlockSpec((BR, D), lambda i: (i, 0)),
            pl.BlockSpec((BR, D), lambda i: (i, 0)),
            pl.BlockSpec((BR, MSG), lambda i: (i, 0)),
            pl.BlockSpec((BR, 1), lambda i: (i, 0)),
            pl.BlockSpec((BR, 1), lambda i: (i, 0)),
            pl.BlockSpec((BR, D), lambda i: (i, 0)),
            pl.BlockSpec((MSG, D), lambda i: (0, 0)),
            pl.BlockSpec((TD, D), lambda i: (0, 0)),
            pl.BlockSpec((1, D), lambda i: (0, 0)),
            pl.BlockSpec((1, TD), lambda i: (0, 0)),
        ],
        out_specs=pl.BlockSpec((BR, D), lambda i: (i, 0)),
        out_shape=jax.ShapeDtypeStruct((N, D), jnp.float32),
    )(A, B2g, msg_g, dtw, valid, memory, Wu3, Wu4, b_upd, w_time)


def kernel(edge_index, t, msg, memory, last_update, W_self, W_nbr, b, W_upd, b_upd, w_time):
    src, dst = edge_index[0], edge_index[1]
    Wcat = jnp.concatenate([W_self, W_upd[:D], W_upd[D:2 * D]], axis=1)
    Wu3 = W_upd[2 * D:2 * D + MSG]
    Wu4 = W_upd[2 * D + MSG:]

    S = jnp.zeros((N, D), jnp.float32).at[dst].add(memory[src])

    H, A, B2 = _dense_phase(memory, S, Wcat, W_nbr, b[None, :])

    pad_ids = (jnp.arange(E_PAD - E, dtype=jnp.int32) * 7919) % N
    src_p = jnp.concatenate([src, pad_ids])
    dst_p = jnp.concatenate([dst, pad_ids])
    idx_all = jnp.concatenate([src_p, dst_p])
    hsd = _gather_rows(H, idx_all, D)
    h_src = hsd[:E]
    h_dst = hsd[E_PAD:E_PAD + E]

    pad_oob = N + (jnp.arange(E_PAD - E, dtype=jnp.int32) % (N_PAD - N))
    src_ob = jnp.concatenate([src, pad_oob])
    dst_ob = jnp.concatenate([dst, pad_oob])
    p2 = _winner(src_ob, dst_ob)
    p1 = jnp.maximum(p2[0, :N], p2[1, :N])

    valid = (p1 > 0).astype(jnp.int32)
    is_dst = p1 > E_PAD
    ew = jnp.where(is_dst, p1 - 1 - E_PAD, p1 - 1)
    spread = jnp.arange(N, dtype=jnp.int32) % E
    ew_c = jnp.where(p1 > 0, jnp.clip(ew, 0, E - 1), spread)
    opp = jnp.where(is_dst, src[ew_c], dst[ew_c])
    dtw = t[ew_c] - last_update
    new_lu = jnp.where(valid > 0, t[ew_c], last_update)
    msg_g = msg[ew_c]
    B2g = B2[opp]

    new_memory = _final_phase(A, B2g, msg_g, dtw[:, None], valid[:, None],
                              memory, Wu3, Wu4, b_upd[None, :], w_time[None, :])

    return (h_src, h_dst, new_memory, new_lu)


# speedup vs baseline: 3.3960x; 1.2370x over previous
import functools

import jax
import jax.numpy as jnp
from jax import lax
from jax.experimental import pallas as pl
from jax.experimental.pallas import tpu as pltpu
from jax.experimental.pallas import tpu_sc as plsc

N = 100000
D = 128
E = 100000
MSG = 16
TD = 16

BR = 2000

NC = 2
NS = 16
NW = NC * NS
E_PAD = 102400

_SC_MESH = plsc.VectorSubcoreMesh(core_axis_name="c", subcore_axis_name="s")


def _gather_rows(table, idx, d, kb=320):
    m = idx.shape[0]
    pw = m // NW
    nb = pw // kb
    assert m % (NW * kb) == 0 and nb % 2 == 0

    @functools.partial(
        pl.kernel,
        out_type=jax.ShapeDtypeStruct((m, d), jnp.float32),
        mesh=_SC_MESH,
        scratch_types=[
            pltpu.VMEM((pw,), jnp.int32),
            pltpu.VMEM((kb, d), jnp.float32),
            pltpu.VMEM((kb, d), jnp.float32),
            pltpu.SemaphoreType.DMA,
            pltpu.SemaphoreType.DMA,
        ],
    )
    def gk(idx_hbm, tbl_hbm, out_hbm, idx_v, buf0, buf1, sem0, sem1):
        wid = lax.axis_index("s") * NC + lax.axis_index("c")
        base = wid * pw
        pltpu.sync_copy(idx_hbm.at[pl.ds(base, pw)], idx_v)
        pltpu.async_copy(tbl_hbm.at[idx_v.at[pl.ds(0, kb)]], buf0, sem0)

        @pl.loop(0, nb, step=2)
        def _(b):
            pltpu.make_async_copy(tbl_hbm.at[pl.ds(0, kb)], buf0, sem0).wait()
            pltpu.async_copy(tbl_hbm.at[idx_v.at[pl.ds((b + 1) * kb, kb)]],
                             buf1, sem1)
            pltpu.sync_copy(buf0, out_hbm.at[pl.ds(base + b * kb, kb)])

            @pl.when(b + 2 < nb)
            def _():
                pltpu.async_copy(tbl_hbm.at[idx_v.at[pl.ds((b + 2) * kb, kb)]],
                                 buf0, sem0)

            pltpu.make_async_copy(tbl_hbm.at[pl.ds(0, kb)], buf1, sem1).wait()
            pltpu.sync_copy(buf1, out_hbm.at[pl.ds(base + (b + 1) * kb, kb)])

    return gk(idx, table)


N_PAD = 102400


def _winner(src_p, dst_p):
    epw = E_PAD // NW
    ngrp = epw // 16
    CHK = 10240
    NR = N_PAD // CHK
    mpt = CHK // NS

    @functools.partial(
        pl.kernel,
        out_type=jax.ShapeDtypeStruct((2, N_PAD), jnp.int32),
        mesh=_SC_MESH,
        compiler_params=pltpu.CompilerParams(needs_layout_passes=False),
        scratch_types=[
            pltpu.VMEM((epw,), jnp.int32),
            pltpu.VMEM((epw,), jnp.int32),
            pltpu.VMEM((N_PAD,), jnp.int32),
            pltpu.VMEM((NS, mpt), jnp.int32),
            pltpu.VMEM((mpt,), jnp.int32),
            pltpu.VMEM_SHARED((NS, CHK), jnp.int32),
        ],
    )
    def wk(src_hbm, dst_hbm, out_hbm, sbuf, dbuf, win, mbuf, tmp, shared):
        cid = lax.axis_index("c")
        sid = lax.axis_index("s")
        wid = sid * NC + cid
        ebase = wid * epw
        pltpu.sync_copy(src_hbm.at[pl.ds(ebase, epw)], sbuf)
        pltpu.sync_copy(dst_hbm.at[pl.ds(ebase, epw)], dbuf)

        @pl.loop(0, N_PAD // 16)
        def _(i):
            win[pl.ds(i * 16, 16)] = jnp.zeros((16,), jnp.int32)

        lanes = lax.iota(jnp.int32, 16)

        @pl.loop(0, ngrp)
        def _(g):
            pri0 = ebase + g * 16 + lanes + 1
            for buf, off in ((sbuf, 0), (dbuf, E_PAD)):
                ids = buf[pl.ds(g * 16, 16)]
                pri = pri0 + off

                def attempt(_):
                    cur = plsc.load_gather(win, [ids])
                    plsc.store_scatter(win, [ids], jnp.maximum(cur, pri))
                    cur2 = plsc.load_gather(win, [ids])
                    return jnp.any(cur2 < pri)

                lax.while_loop(lambda need: need, attempt, attempt(None))

        for k in range(NR):
            pltpu.sync_copy(win.at[pl.ds(k * CHK, CHK)], shared.at[sid])
            plsc.subcore_barrier()
            pltpu.sync_copy(shared.at[:, pl.ds(sid * mpt, mpt)], mbuf)

            @pl.loop(0, mpt // 16)
            def _(g):
                sl = pl.ds(g * 16, 16)
                acc = mbuf[0, sl]
                for r in range(1, NS):
                    acc = jnp.maximum(acc, mbuf[r, sl])
                tmp[sl] = acc

            pltpu.sync_copy(tmp, out_hbm.at[cid, pl.ds(k * CHK + sid * mpt, mpt)])
            plsc.subcore_barrier()

    return wk(src_p, dst_p)


def _scatter_add(src_ob, dst_ob, mem_pad, zrows):
    CH = 10240
    NCHUNK = N_PAD // CH
    ZR = CH // NS + 16
    SROWS = NS * ZR
    epc = E_PAD // NS
    GB = 128
    CAP = epc + 2 * GB

    @functools.partial(
        pl.kernel,
        out_type=jax.ShapeDtypeStruct((N_PAD, D), jnp.float32),
        mesh=_SC_MESH,
        compiler_params=pltpu.CompilerParams(needs_layout_passes=False),
        scratch_types=[
            pltpu.VMEM((CAP,), jnp.int32),
            pltpu.VMEM((CAP,), jnp.int32),
            pltpu.VMEM((GB, D), jnp.float32),
            pltpu.VMEM((GB,), jnp.int32),
            pltpu.VMEM((GB,), jnp.int32),
            pltpu.VMEM_SHARED((SROWS, D), jnp.float32),
            pltpu.SemaphoreType.DMA,
        ],
    )
    def sk(src_hbm, dst_hbm, mem_hbm, z_hbm, out_hbm,
           sbuf, dbuf, rows, idxg, lidg, sloc, semg):
        cid = lax.axis_index("c")
        sid = lax.axis_index("s")
        lanes = lax.iota(jnp.int32, 16)

        for kc in range(NCHUNK // NC):
            lo = (NC * kc + cid) * CH
            pltpu.sync_copy(z_hbm, sloc.at[pl.ds(sid * ZR, ZR)])
            pltpu.sync_copy(src_hbm.at[pl.ds(sid * epc, epc)], sbuf.at[pl.ds(0, epc)])
            pltpu.sync_copy(dst_hbm.at[pl.ds(sid * epc, epc)], dbuf.at[pl.ds(0, epc)])
            plsc.subcore_barrier()

            @pl.loop(0, epc // 16, init_carry=jnp.int32(0))
            def compact(g, off):
                ids_d = dbuf[pl.ds(g * 16, 16)]
                ids_s = sbuf[pl.ds(g * 16, 16)]
                lid = ids_d - lo
                mask = (lid >= 0) & (lid < CH)
                plsc.store_compressed(dbuf.at[pl.ds(off, 16)], lid, mask=mask)
                plsc.store_compressed(sbuf.at[pl.ds(off, 16)], ids_s, mask=mask)
                return off + jnp.sum(mask.astype(jnp.int32))

            off = compact
            for j in range(GB // 16):
                dbuf[pl.ds(off + j * 16, 16)] = CH + lanes
                sbuf[pl.ds(off + j * 16, 16)] = lanes

            nbb = (off + GB - 1) // GB

            @pl.loop(0, nbb)
            def gather_scatter(b):
                for k in range(GB // 16):
                    sl16 = pl.ds(k * 16, 16)
                    idxg[sl16] = sbuf[pl.ds(b * GB + k * 16, 16)]
                    lidg[sl16] = dbuf[pl.ds(b * GB + k * 16, 16)]
                pltpu.async_copy(mem_hbm.at[idxg], rows, semg).wait()
                pltpu.sync_copy(rows, sloc.at[lidg], add=True)

            plsc.subcore_barrier()
            wb = CH // NS
            pltpu.sync_copy(sloc.at[pl.ds(sid * wb, wb)],
                            out_hbm.at[pl.ds(lo + sid * wb, wb)])
            plsc.subcore_barrier()

    return sk(src_ob, dst_ob, mem_pad, zrows)


def _dense_body(mem_ref, s_ref, wcat_ref, wnbr_ref, b_ref, h_ref, a_ref, b2_ref):
    mem = mem_ref[...]
    c3 = jnp.dot(mem, wcat_ref[...], preferred_element_type=jnp.float32)
    hpre = c3[:, :D] + jnp.dot(s_ref[...], wnbr_ref[...],
                               preferred_element_type=jnp.float32) + b_ref[...]
    h_ref[...] = jnp.tanh(hpre)
    a_ref[...] = c3[:, D:2 * D]
    b2_ref[...] = c3[:, 2 * D:3 * D]


def _dense_phase(memory, S, Wcat, W_nbr, b):
    grid = (N // BR,)
    return pl.pallas_call(
        _dense_body,
        grid=grid,
        in_specs=[
            pl.BlockSpec((BR, D), lambda i: (i, 0)),
            pl.BlockSpec((BR, D), lambda i: (i, 0)),
            pl.BlockSpec((D, 3 * D), lambda i: (0, 0)),
            pl.BlockSpec((D, D), lambda i: (0, 0)),
            pl.BlockSpec((1, D), lambda i: (0, 0)),
        ],
        out_specs=[
            pl.BlockSpec((BR, D), lambda i: (i, 0)),
            pl.BlockSpec((BR, D), lambda i: (i, 0)),
            pl.BlockSpec((BR, D), lambda i: (i, 0)),
        ],
        out_shape=[jax.ShapeDtypeStruct((N, D), jnp.float32)] * 3,
    )(memory, S, Wcat, W_nbr, b)


def _final_body(a_ref, b2g_ref, msgg_ref, dtw_ref, valid_ref, mem_ref,
                wu3_ref, wu4_ref, bupd_ref, wt_ref, out_ref):
    te = jnp.cos(dtw_ref[...] * wt_ref[...])
    pre = (a_ref[...] + b2g_ref[...]
           + jnp.dot(msgg_ref[...], wu3_ref[...], preferred_element_type=jnp.float32)
           + jnp.dot(te, wu4_ref[...], preferred_element_type=jnp.float32)
           + bupd_ref[...])
    m = jnp.tanh(pre)
    out_ref[...] = jnp.where(valid_ref[...] > 0, m, mem_ref[...])


def _final_phase(A, B2g, msg_g, dtw, valid, memory, Wu3, Wu4, b_upd, w_time):
    grid = (N // BR,)
    return pl.pallas_call(
        _final_body,
        grid=grid,
        in_specs=[
            pl.BlockSpec((BR, D), lambda i: (i, 0)),
            pl.BlockSpec((BR, D), lambda i: (i, 0)),
            pl.BlockSpec((BR, MSG), lambda i: (i, 0)),
            pl.BlockSpec((BR, 1), lambda i: (i, 0)),
            pl.BlockSpec((BR, 1), lambda i: (i, 0)),
            pl.BlockSpec((BR, D), lambda i: (i, 0)),
            pl.BlockSpec((MSG, D), lambda i: (0, 0)),
            pl.BlockSpec((TD, D), lambda i: (0, 0)),
            pl.BlockSpec((1, D), lambda i: (0, 0)),
            pl.BlockSpec((1, TD), lambda i: (0, 0)),
        ],
        out_specs=pl.BlockSpec((BR, D), lambda i: (i, 0)),
        out_shape=jax.ShapeDtypeStruct((N, D), jnp.float32),
    )(A, B2g, msg_g, dtw, valid, memory, Wu3, Wu4, b_upd, w_time)


def kernel(edge_index, t, msg, memory, last_update, W_self, W_nbr, b, W_upd, b_upd, w_time):
    src, dst = edge_index[0], edge_index[1]
    Wcat = jnp.concatenate([W_self, W_upd[:D], W_upd[D:2 * D]], axis=1)
    Wu3 = W_upd[2 * D:2 * D + MSG]
    Wu4 = W_upd[2 * D + MSG:]

    pad_oob0 = N + (jnp.arange(E_PAD - E, dtype=jnp.int32) % (N_PAD - N))
    src_ob0 = jnp.concatenate([src, pad_oob0])
    dst_ob0 = jnp.concatenate([dst, pad_oob0])
    mem_pad = jnp.pad(memory, ((0, N_PAD - N), (0, 0)))
    zrows = jnp.zeros((656, D), jnp.float32)
    S_pad = _scatter_add(src_ob0, dst_ob0, mem_pad, zrows)

    H, A, B2 = _dense_phase(memory, S_pad, Wcat, W_nbr, b[None, :])

    pad_ids = (jnp.arange(E_PAD - E, dtype=jnp.int32) * 7919) % N
    src_p = jnp.concatenate([src, pad_ids])
    dst_p = jnp.concatenate([dst, pad_ids])
    idx_all = jnp.concatenate([src_p, dst_p])
    hsd = _gather_rows(H, idx_all, D)
    h_src = hsd[:E]
    h_dst = hsd[E_PAD:E_PAD + E]

    pad_oob = N + (jnp.arange(E_PAD - E, dtype=jnp.int32) % (N_PAD - N))
    src_ob = jnp.concatenate([src, pad_oob])
    dst_ob = jnp.concatenate([dst, pad_oob])
    p2 = _winner(src_ob, dst_ob)
    p1 = jnp.maximum(p2[0, :N], p2[1, :N])

    valid = (p1 > 0).astype(jnp.int32)
    is_dst = p1 > E_PAD
    ew = jnp.where(is_dst, p1 - 1 - E_PAD, p1 - 1)
    spread = jnp.arange(N, dtype=jnp.int32) % E
    ew_c = jnp.where(p1 > 0, jnp.clip(ew, 0, E - 1), spread)
    opp = jnp.where(is_dst, src[ew_c], dst[ew_c])
    dtw = t[ew_c] - last_update
    new_lu = jnp.where(valid > 0, t[ew_c], last_update)
    msg_g = msg[ew_c]
    B2g = B2[opp]

    new_memory = _final_phase(A, B2g, msg_g, dtw[:, None], valid[:, None],
                              memory, Wu3, Wu4, b_upd[None, :], w_time[None, :])

    return (h_src, h_dst, new_memory, new_lu)


# speedup vs baseline: 3.4739x; 1.0230x over previous
import functools

import jax
import jax.numpy as jnp
from jax import lax
from jax.experimental import pallas as pl
from jax.experimental.pallas import tpu as pltpu
from jax.experimental.pallas import tpu_sc as plsc

N = 100000
D = 128
E = 100000
MSG = 16
TD = 16

BR = 2000

NC = 2
NS = 16
NW = NC * NS
E_PAD = 102400

_SC_MESH = plsc.VectorSubcoreMesh(core_axis_name="c", subcore_axis_name="s")


def _gather_rows(table, idx, d, kb=320):
    m = idx.shape[0]
    pw = m // NW
    nb = pw // kb
    assert m % (NW * kb) == 0 and nb % 2 == 0

    @functools.partial(
        pl.kernel,
        out_type=jax.ShapeDtypeStruct((m, d), jnp.float32),
        mesh=_SC_MESH,
        scratch_types=[
            pltpu.VMEM((pw,), jnp.int32),
            pltpu.VMEM((kb, d), jnp.float32),
            pltpu.VMEM((kb, d), jnp.float32),
            pltpu.SemaphoreType.DMA,
            pltpu.SemaphoreType.DMA,
        ],
    )
    def gk(idx_hbm, tbl_hbm, out_hbm, idx_v, buf0, buf1, sem0, sem1):
        wid = lax.axis_index("s") * NC + lax.axis_index("c")
        base = wid * pw
        pltpu.sync_copy(idx_hbm.at[pl.ds(base, pw)], idx_v)
        pltpu.async_copy(tbl_hbm.at[idx_v.at[pl.ds(0, kb)]], buf0, sem0)

        @pl.loop(0, nb, step=2)
        def _(b):
            pltpu.make_async_copy(tbl_hbm.at[pl.ds(0, kb)], buf0, sem0).wait()
            pltpu.async_copy(tbl_hbm.at[idx_v.at[pl.ds((b + 1) * kb, kb)]],
                             buf1, sem1)
            pltpu.sync_copy(buf0, out_hbm.at[pl.ds(base + b * kb, kb)])

            @pl.when(b + 2 < nb)
            def _():
                pltpu.async_copy(tbl_hbm.at[idx_v.at[pl.ds((b + 2) * kb, kb)]],
                                 buf0, sem0)

            pltpu.make_async_copy(tbl_hbm.at[pl.ds(0, kb)], buf1, sem1).wait()
            pltpu.sync_copy(buf1, out_hbm.at[pl.ds(base + (b + 1) * kb, kb)])

    return gk(idx, table)


N_PAD = 102400


def _winner(src_p, dst_p):
    epw = E_PAD // NW
    ngrp = epw // 16
    CHK = 10240
    NR = N_PAD // CHK
    mpt = CHK // NS

    @functools.partial(
        pl.kernel,
        out_type=jax.ShapeDtypeStruct((2, N_PAD), jnp.int32),
        mesh=_SC_MESH,
        compiler_params=pltpu.CompilerParams(needs_layout_passes=False),
        scratch_types=[
            pltpu.VMEM((epw,), jnp.int32),
            pltpu.VMEM((epw,), jnp.int32),
            pltpu.VMEM((N_PAD,), jnp.int32),
            pltpu.VMEM((NS, mpt), jnp.int32),
            pltpu.VMEM((mpt,), jnp.int32),
            pltpu.VMEM_SHARED((NS, CHK), jnp.int32),
        ],
    )
    def wk(src_hbm, dst_hbm, out_hbm, sbuf, dbuf, win, mbuf, tmp, shared):
        cid = lax.axis_index("c")
        sid = lax.axis_index("s")
        wid = sid * NC + cid
        ebase = wid * epw
        pltpu.sync_copy(src_hbm.at[pl.ds(ebase, epw)], sbuf)
        pltpu.sync_copy(dst_hbm.at[pl.ds(ebase, epw)], dbuf)

        @pl.loop(0, N_PAD // 16)
        def _(i):
            win[pl.ds(i * 16, 16)] = jnp.zeros((16,), jnp.int32)

        lanes = lax.iota(jnp.int32, 16)

        @pl.loop(0, ngrp)
        def _(g):
            pri0 = ebase + g * 16 + lanes + 1
            for buf, off in ((sbuf, 0), (dbuf, E_PAD)):
                ids = buf[pl.ds(g * 16, 16)]
                pri = pri0 + off

                def attempt(_):
                    cur = plsc.load_gather(win, [ids])
                    plsc.store_scatter(win, [ids], jnp.maximum(cur, pri))
                    cur2 = plsc.load_gather(win, [ids])
                    return jnp.any(cur2 < pri)

                lax.while_loop(lambda need: need, attempt, attempt(None))

        for k in range(NR):
            pltpu.sync_copy(win.at[pl.ds(k * CHK, CHK)], shared.at[sid])
            plsc.subcore_barrier()
            pltpu.sync_copy(shared.at[:, pl.ds(sid * mpt, mpt)], mbuf)

            @pl.loop(0, mpt // 16)
            def _(g):
                sl = pl.ds(g * 16, 16)
                acc = mbuf[0, sl]
                for r in range(1, NS):
                    acc = jnp.maximum(acc, mbuf[r, sl])
                tmp[sl] = acc

            pltpu.sync_copy(tmp, out_hbm.at[cid, pl.ds(k * CHK + sid * mpt, mpt)])
            plsc.subcore_barrier()

    return wk(src_p, dst_p)


def _scatter_add(src_ob, dst_ob, mem_pad, zrows):
    CH = 10240
    NCHUNK = N_PAD // CH
    ZR = CH // NS + 16
    SROWS = NS * ZR
    epc = E_PAD // NS
    GB = 128
    CAP = epc + 2 * GB

    @functools.partial(
        pl.kernel,
        out_type=jax.ShapeDtypeStruct((N_PAD, D), jnp.float32),
        mesh=_SC_MESH,
        compiler_params=pltpu.CompilerParams(needs_layout_passes=False),
        scratch_types=[
            pltpu.VMEM((CAP,), jnp.int32),
            pltpu.VMEM((CAP,), jnp.int32),
            pltpu.VMEM((GB, D), jnp.float32),
            pltpu.VMEM((GB,), jnp.int32),
            pltpu.VMEM((GB,), jnp.int32),
            pltpu.VMEM_SHARED((SROWS, D), jnp.float32),
            pltpu.SemaphoreType.DMA,
        ],
    )
    def sk(src_hbm, dst_hbm, mem_hbm, z_hbm, out_hbm,
           sbuf, dbuf, rows, idxg, lidg, sloc, semg):
        cid = lax.axis_index("c")
        sid = lax.axis_index("s")
        lanes = lax.iota(jnp.int32, 16)

        for kc in range(NCHUNK // NC):
            lo = (NC * kc + cid) * CH
            pltpu.sync_copy(z_hbm, sloc.at[pl.ds(sid * ZR, ZR)])
            pltpu.sync_copy(src_hbm.at[pl.ds(sid * epc, epc)], sbuf.at[pl.ds(0, epc)])
            pltpu.sync_copy(dst_hbm.at[pl.ds(sid * epc, epc)], dbuf.at[pl.ds(0, epc)])
            plsc.subcore_barrier()

            @pl.loop(0, epc // 16, init_carry=jnp.int32(0))
            def compact(g, off):
                ids_d = dbuf[pl.ds(g * 16, 16)]
                ids_s = sbuf[pl.ds(g * 16, 16)]
                lid = ids_d - lo
                mask = (lid >= 0) & (lid < CH)
                plsc.store_compressed(dbuf.at[pl.ds(off, 16)], lid, mask=mask)
                plsc.store_compressed(sbuf.at[pl.ds(off, 16)], ids_s, mask=mask)
                return off + jnp.sum(mask.astype(jnp.int32))

            off = compact
            for j in range(GB // 16):
                dbuf[pl.ds(off + j * 16, 16)] = CH + lanes
                sbuf[pl.ds(off + j * 16, 16)] = lanes

            nbb = (off + GB - 1) // GB

            @pl.loop(0, nbb)
            def gather_scatter(b):
                for k in range(GB // 16):
                    sl16 = pl.ds(k * 16, 16)
                    idxg[sl16] = sbuf[pl.ds(b * GB + k * 16, 16)]
                    lidg[sl16] = dbuf[pl.ds(b * GB + k * 16, 16)]
                pltpu.async_copy(mem_hbm.at[idxg], rows, semg).wait()
                pltpu.sync_copy(rows, sloc.at[lidg], add=True)

            plsc.subcore_barrier()
            wb = CH // NS
            pltpu.sync_copy(sloc.at[pl.ds(sid * wb, wb)],
                            out_hbm.at[pl.ds(lo + sid * wb, wb)])
            plsc.subcore_barrier()

    return sk(src_ob, dst_ob, mem_pad, zrows)


def _dense_body(mem_ref, s_ref, wcat_ref, wnbr_ref, b_ref, h_ref, b2_ref):
    mem = mem_ref[...]
    c2 = jnp.dot(mem, wcat_ref[...], preferred_element_type=jnp.float32)
    hpre = c2[:, :D] + jnp.dot(s_ref[...], wnbr_ref[...],
                               preferred_element_type=jnp.float32) + b_ref[...]
    h_ref[...] = jnp.tanh(hpre)
    b2_ref[...] = c2[:, D:2 * D]


def _dense_phase(memory, S, Wcat, W_nbr, b):
    grid = (N // BR,)
    return pl.pallas_call(
        _dense_body,
        grid=grid,
        in_specs=[
            pl.BlockSpec((BR, D), lambda i: (i, 0)),
            pl.BlockSpec((BR, D), lambda i: (i, 0)),
            pl.BlockSpec((D, 2 * D), lambda i: (0, 0)),
            pl.BlockSpec((D, D), lambda i: (0, 0)),
            pl.BlockSpec((1, D), lambda i: (0, 0)),
        ],
        out_specs=[
            pl.BlockSpec((BR, D), lambda i: (i, 0)),
            pl.BlockSpec((BR, D), lambda i: (i, 0)),
        ],
        out_shape=[jax.ShapeDtypeStruct((N, D), jnp.float32)] * 2,
    )(memory, S, Wcat, W_nbr, b)


def _final_body(wu1_ref, b2g_ref, msgg_ref, dtw_ref, valid_ref, mem_ref,
                wu3_ref, wu4_ref, bupd_ref, wt_ref, out_ref):
    te = jnp.cos(dtw_ref[...] * wt_ref[...])
    a_blk = jnp.dot(mem_ref[...], wu1_ref[...], preferred_element_type=jnp.float32)
    pre = (a_blk + b2g_ref[...]
           + jnp.dot(msgg_ref[...], wu3_ref[...], preferred_element_type=jnp.float32)
           + jnp.dot(te, wu4_ref[...], preferred_element_type=jnp.float32)
           + bupd_ref[...])
    m = jnp.tanh(pre)
    out_ref[...] = jnp.where(valid_ref[...] > 0, m, mem_ref[...])


def _final_phase(Wu1, B2g, msg_g, dtw, valid, memory, Wu3, Wu4, b_upd, w_time):
    grid = (N // BR,)
    return pl.pallas_call(
        _final_body,
        grid=grid,
        in_specs=[
            pl.BlockSpec((D, D), lambda i: (0, 0)),
            pl.BlockSpec((BR, D), lambda i: (i, 0)),
            pl.BlockSpec((BR, MSG), lambda i: (i, 0)),
            pl.BlockSpec((BR, 1), lambda i: (i, 0)),
            pl.BlockSpec((BR, 1), lambda i: (i, 0)),
            pl.BlockSpec((BR, D), lambda i: (i, 0)),
            pl.BlockSpec((MSG, D), lambda i: (0, 0)),
            pl.BlockSpec((TD, D), lambda i: (0, 0)),
            pl.BlockSpec((1, D), lambda i: (0, 0)),
            pl.BlockSpec((1, TD), lambda i: (0, 0)),
        ],
        out_specs=pl.BlockSpec((BR, D), lambda i: (i, 0)),
        out_shape=jax.ShapeDtypeStruct((N, D), jnp.float32),
    )(Wu1, B2g, msg_g, dtw, valid, memory, Wu3, Wu4, b_upd, w_time)


def kernel(edge_index, t, msg, memory, last_update, W_self, W_nbr, b, W_upd, b_upd, w_time):
    src, dst = edge_index[0], edge_index[1]
    Wcat = jnp.concatenate([W_self, W_upd[D:2 * D]], axis=1)
    Wu3 = W_upd[2 * D:2 * D + MSG]
    Wu4 = W_upd[2 * D + MSG:]

    pad_oob0 = N + (jnp.arange(E_PAD - E, dtype=jnp.int32) % (N_PAD - N))
    src_ob0 = jnp.concatenate([src, pad_oob0])
    dst_ob0 = jnp.concatenate([dst, pad_oob0])
    mem_pad = jnp.pad(memory, ((0, N_PAD - N), (0, 0)))
    zrows = jnp.zeros((656, D), jnp.float32)
    S_pad = _scatter_add(src_ob0, dst_ob0, mem_pad, zrows)

    H, B2 = _dense_phase(memory, S_pad, Wcat, W_nbr, b[None, :])

    pad_ids = (jnp.arange(E_PAD - E, dtype=jnp.int32) * 7919) % N
    src_p = jnp.concatenate([src, pad_ids])
    dst_p = jnp.concatenate([dst, pad_ids])
    idx_all = jnp.concatenate([src_p, dst_p])
    hsd = _gather_rows(H, idx_all, D)
    h_src = hsd[:E]
    h_dst = hsd[E_PAD:E_PAD + E]

    pad_oob = N + (jnp.arange(E_PAD - E, dtype=jnp.int32) % (N_PAD - N))
    src_ob = jnp.concatenate([src, pad_oob])
    dst_ob = jnp.concatenate([dst, pad_oob])
    p2 = _winner(src_ob, dst_ob)
    p1 = jnp.maximum(p2[0, :N], p2[1, :N])

    valid = (p1 > 0).astype(jnp.int32)
    is_dst = p1 > E_PAD
    ew = jnp.where(is_dst, p1 - 1 - E_PAD, p1 - 1)
    spread = jnp.arange(N, dtype=jnp.int32) % E
    ew_c = jnp.where(p1 > 0, jnp.clip(ew, 0, E - 1), spread)
    opp = jnp.where(is_dst, src[ew_c], dst[ew_c])
    dtw = t[ew_c] - last_update
    new_lu = jnp.where(valid > 0, t[ew_c], last_update)
    msg_g = msg[ew_c]
    B2g = B2[opp]

    new_memory = _final_phase(W_upd[:D], B2g, msg_g, dtw[:, None], valid[:, None],
                              memory, Wu3, Wu4, b_upd[None, :], w_time[None, :])

    return (h_src, h_dst, new_memory, new_lu)


# speedup vs baseline: 3.6228x; 1.0429x over previous
import functools

import jax
import jax.numpy as jnp
from jax import lax
from jax.experimental import pallas as pl
from jax.experimental.pallas import tpu as pltpu
from jax.experimental.pallas import tpu_sc as plsc

N = 100000
D = 128
E = 100000
MSG = 16
TD = 16

BR = 2000

NC = 2
NS = 16
NW = NC * NS
E_PAD = 102400

_SC_MESH = plsc.VectorSubcoreMesh(core_axis_name="c", subcore_axis_name="s")


def _gather_rows(table, idx, d, kb=320):
    m = idx.shape[0]
    pw = m // NW
    nb = pw // kb
    assert m % (NW * kb) == 0 and nb % 2 == 0

    @functools.partial(
        pl.kernel,
        out_type=jax.ShapeDtypeStruct((m, d), jnp.float32),
        mesh=_SC_MESH,
        scratch_types=[
            pltpu.VMEM((pw,), jnp.int32),
            pltpu.VMEM((kb, d), jnp.float32),
            pltpu.VMEM((kb, d), jnp.float32),
            pltpu.SemaphoreType.DMA,
            pltpu.SemaphoreType.DMA,
        ],
    )
    def gk(idx_hbm, tbl_hbm, out_hbm, idx_v, buf0, buf1, sem0, sem1):
        wid = lax.axis_index("s") * NC + lax.axis_index("c")
        base = wid * pw
        pltpu.sync_copy(idx_hbm.at[pl.ds(base, pw)], idx_v)
        pltpu.async_copy(tbl_hbm.at[idx_v.at[pl.ds(0, kb)]], buf0, sem0)

        @pl.loop(0, nb, step=2)
        def _(b):
            pltpu.make_async_copy(tbl_hbm.at[pl.ds(0, kb)], buf0, sem0).wait()
            pltpu.async_copy(tbl_hbm.at[idx_v.at[pl.ds((b + 1) * kb, kb)]],
                             buf1, sem1)
            pltpu.sync_copy(buf0, out_hbm.at[pl.ds(base + b * kb, kb)])

            @pl.when(b + 2 < nb)
            def _():
                pltpu.async_copy(tbl_hbm.at[idx_v.at[pl.ds((b + 2) * kb, kb)]],
                                 buf0, sem0)

            pltpu.make_async_copy(tbl_hbm.at[pl.ds(0, kb)], buf1, sem1).wait()
            pltpu.sync_copy(buf1, out_hbm.at[pl.ds(base + (b + 1) * kb, kb)])

    return gk(idx, table)


N_PAD = 102400


def _winner(src_p, dst_p):
    epw = E_PAD // NW
    ngrp = epw // 16
    CHK = 10240
    NR = N_PAD // CHK
    mpt = CHK // NS

    @functools.partial(
        pl.kernel,
        out_type=jax.ShapeDtypeStruct((2, N_PAD), jnp.int32),
        mesh=_SC_MESH,
        compiler_params=pltpu.CompilerParams(needs_layout_passes=False),
        scratch_types=[
            pltpu.VMEM((epw,), jnp.int32),
            pltpu.VMEM((epw,), jnp.int32),
            pltpu.VMEM((N_PAD,), jnp.int32),
            pltpu.VMEM((NS, mpt), jnp.int32),
            pltpu.VMEM((mpt,), jnp.int32),
            pltpu.VMEM_SHARED((NS, CHK), jnp.int32),
        ],
    )
    def wk(src_hbm, dst_hbm, out_hbm, sbuf, dbuf, win, mbuf, tmp, shared):
        cid = lax.axis_index("c")
        sid = lax.axis_index("s")
        wid = sid * NC + cid
        ebase = wid * epw
        pltpu.sync_copy(src_hbm.at[pl.ds(ebase, epw)], sbuf)
        pltpu.sync_copy(dst_hbm.at[pl.ds(ebase, epw)], dbuf)

        @pl.loop(0, N_PAD // 16)
        def _(i):
            win[pl.ds(i * 16, 16)] = jnp.zeros((16,), jnp.int32)

        lanes = lax.iota(jnp.int32, 16)

        @pl.loop(0, ngrp)
        def _(g):
            pri0 = ebase + g * 16 + lanes + 1
            for buf, off in ((sbuf, 0), (dbuf, E_PAD)):
                ids = buf[pl.ds(g * 16, 16)]
                pri = pri0 + off

                def attempt(_):
                    cur = plsc.load_gather(win, [ids])
                    plsc.store_scatter(win, [ids], jnp.maximum(cur, pri))
                    cur2 = plsc.load_gather(win, [ids])
                    return jnp.any(cur2 < pri)

                lax.while_loop(lambda need: need, attempt, attempt(None))

        for k in range(NR):
            pltpu.sync_copy(win.at[pl.ds(k * CHK, CHK)], shared.at[sid])
            plsc.subcore_barrier()
            pltpu.sync_copy(shared.at[:, pl.ds(sid * mpt, mpt)], mbuf)

            @pl.loop(0, mpt // 16)
            def _(g):
                sl = pl.ds(g * 16, 16)
                acc = mbuf[0, sl]
                for r in range(1, NS):
                    acc = jnp.maximum(acc, mbuf[r, sl])
                tmp[sl] = acc

            pltpu.sync_copy(tmp, out_hbm.at[cid, pl.ds(k * CHK + sid * mpt, mpt)])
            plsc.subcore_barrier()

    return wk(src_p, dst_p)


def _scatter_add(src_ob, dst_ob, mem_pad, zrows):
    CH = 10240
    NCHUNK = N_PAD // CH
    ZR = CH // NS + 16
    SROWS = NS * ZR
    epc = E_PAD // NS
    GB = 128
    CAP = epc + 2 * GB

    @functools.partial(
        pl.kernel,
        out_type=jax.ShapeDtypeStruct((N_PAD, D), jnp.float32),
        mesh=_SC_MESH,
        compiler_params=pltpu.CompilerParams(needs_layout_passes=False),
        scratch_types=[
            pltpu.VMEM((CAP,), jnp.int32),
            pltpu.VMEM((CAP,), jnp.int32),
            pltpu.VMEM((GB, D), jnp.float32),
            pltpu.VMEM((GB, D), jnp.float32),
            pltpu.VMEM((GB,), jnp.int32),
            pltpu.VMEM((GB,), jnp.int32),
            pltpu.VMEM((GB,), jnp.int32),
            pltpu.VMEM((GB,), jnp.int32),
            pltpu.VMEM_SHARED((SROWS, D), jnp.float32),
            pltpu.SemaphoreType.DMA,
            pltpu.SemaphoreType.DMA,
        ],
    )
    def sk(src_hbm, dst_hbm, mem_hbm, z_hbm, out_hbm,
           sbuf, dbuf, rows0, rows1, idxg0, idxg1, lidg0, lidg1,
           sloc, semg0, semg1):
        cid = lax.axis_index("c")
        sid = lax.axis_index("s")
        lanes = lax.iota(jnp.int32, 16)

        for kc in range(NCHUNK // NC):
            lo = (NC * kc + cid) * CH
            pltpu.sync_copy(z_hbm, sloc.at[pl.ds(sid * ZR, ZR)])
            pltpu.sync_copy(src_hbm.at[pl.ds(sid * epc, epc)], sbuf.at[pl.ds(0, epc)])
            pltpu.sync_copy(dst_hbm.at[pl.ds(sid * epc, epc)], dbuf.at[pl.ds(0, epc)])
            plsc.subcore_barrier()

            @pl.loop(0, epc // 16, init_carry=jnp.int32(0))
            def compact(g, off):
                ids_d = dbuf[pl.ds(g * 16, 16)]
                ids_s = sbuf[pl.ds(g * 16, 16)]
                lid = ids_d - lo
                mask = (lid >= 0) & (lid < CH)
                plsc.store_compressed(dbuf.at[pl.ds(off, 16)], lid, mask=mask)
                plsc.store_compressed(sbuf.at[pl.ds(off, 16)], ids_s, mask=mask)
                return off + jnp.sum(mask.astype(jnp.int32))

            off = compact
            for j in range(2 * GB // 16):
                dbuf[pl.ds(off + j * 16, 16)] = CH + lanes
                sbuf[pl.ds(off + j * 16, 16)] = lanes

            npair = jnp.maximum((off + 2 * GB - 1) // (2 * GB), 1)

            def stage(b, idxg, lidg):
                for k in range(GB // 16):
                    sl16 = pl.ds(k * 16, 16)
                    idxg[sl16] = sbuf[pl.ds(b * GB + k * 16, 16)]
                    lidg[sl16] = dbuf[pl.ds(b * GB + k * 16, 16)]

            stage(jnp.int32(0), idxg0, lidg0)
            pltpu.async_copy(mem_hbm.at[idxg0], rows0, semg0)

            @pl.loop(0, npair)
            def gather_scatter(i):
                b0 = 2 * i
                pltpu.make_async_copy(mem_hbm.at[pl.ds(0, GB)], rows0, semg0).wait()
                stage(b0 + 1, idxg1, lidg1)
                pltpu.async_copy(mem_hbm.at[idxg1], rows1, semg1)
                pltpu.sync_copy(rows0, sloc.at[lidg0], add=True)

                @pl.when(i + 1 < npair)
                def _():
                    stage(b0 + 2, idxg0, lidg0)
                    pltpu.async_copy(mem_hbm.at[idxg0], rows0, semg0)

                pltpu.make_async_copy(mem_hbm.at[pl.ds(0, GB)], rows1, semg1).wait()
                pltpu.sync_copy(rows1, sloc.at[lidg1], add=True)

            plsc.subcore_barrier()
            wb = CH // NS
            pltpu.sync_copy(sloc.at[pl.ds(sid * wb, wb)],
                            out_hbm.at[pl.ds(lo + sid * wb, wb)])
            plsc.subcore_barrier()

    return sk(src_ob, dst_ob, mem_pad, zrows)


def _dense_body(mem_ref, s_ref, wcat_ref, wnbr_ref, b_ref, h_ref, b2_ref):
    mem = mem_ref[...]
    c2 = jnp.dot(mem, wcat_ref[...], preferred_element_type=jnp.float32)
    hpre = c2[:, :D] + jnp.dot(s_ref[...], wnbr_ref[...],
                               preferred_element_type=jnp.float32) + b_ref[...]
    h_ref[...] = jnp.tanh(hpre)
    b2_ref[...] = c2[:, D:2 * D]


def _dense_phase(memory, S, Wcat, W_nbr, b):
    grid = (N // BR,)
    return pl.pallas_call(
        _dense_body,
        grid=grid,
        in_specs=[
            pl.BlockSpec((BR, D), lambda i: (i, 0)),
            pl.BlockSpec((BR, D), lambda i: (i, 0)),
            pl.BlockSpec((D, 2 * D), lambda i: (0, 0)),
            pl.BlockSpec((D, D), lambda i: (0, 0)),
            pl.BlockSpec((1, D), lambda i: (0, 0)),
        ],
        out_specs=[
            pl.BlockSpec((BR, D), lambda i: (i, 0)),
            pl.BlockSpec((BR, D), lambda i: (i, 0)),
        ],
        out_shape=[jax.ShapeDtypeStruct((N, D), jnp.float32)] * 2,
    )(memory, S, Wcat, W_nbr, b)


def _final_body(wu1_ref, b2g_ref, msgg_ref, dtw_ref, valid_ref, mem_ref,
                wu3_ref, wu4_ref, bupd_ref, wt_ref, out_ref):
    te = jnp.cos(dtw_ref[...] * wt_ref[...])
    a_blk = jnp.dot(mem_ref[...], wu1_ref[...], preferred_element_type=jnp.float32)
    pre = (a_blk + b2g_ref[...]
           + jnp.dot(msgg_ref[...], wu3_ref[...], preferred_element_type=jnp.float32)
           + jnp.dot(te, wu4_ref[...], preferred_element_type=jnp.float32)
           + bupd_ref[...])
    m = jnp.tanh(pre)
    out_ref[...] = jnp.where(valid_ref[...] > 0, m, mem_ref[...])


def _final_phase(Wu1, B2g, msg_g, dtw, valid, memory, Wu3, Wu4, b_upd, w_time):
    grid = (N // BR,)
    return pl.pallas_call(
        _final_body,
        grid=grid,
        in_specs=[
            pl.BlockSpec((D, D), lambda i: (0, 0)),
            pl.BlockSpec((BR, D), lambda i: (i, 0)),
            pl.BlockSpec((BR, MSG), lambda i: (i, 0)),
            pl.BlockSpec((BR, 1), lambda i: (i, 0)),
            pl.BlockSpec((BR, 1), lambda i: (i, 0)),
            pl.BlockSpec((BR, D), lambda i: (i, 0)),
            pl.BlockSpec((MSG, D), lambda i: (0, 0)),
            pl.BlockSpec((TD, D), lambda i: (0, 0)),
            pl.BlockSpec((1, D), lambda i: (0, 0)),
            pl.BlockSpec((1, TD), lambda i: (0, 0)),
        ],
        out_specs=pl.BlockSpec((BR, D), lambda i: (i, 0)),
        out_shape=jax.ShapeDtypeStruct((N, D), jnp.float32),
    )(Wu1, B2g, msg_g, dtw, valid, memory, Wu3, Wu4, b_upd, w_time)


def kernel(edge_index, t, msg, memory, last_update, W_self, W_nbr, b, W_upd, b_upd, w_time):
    src, dst = edge_index[0], edge_index[1]
    Wcat = jnp.concatenate([W_self, W_upd[D:2 * D]], axis=1)
    Wu3 = W_upd[2 * D:2 * D + MSG]
    Wu4 = W_upd[2 * D + MSG:]

    pad_oob0 = N + (jnp.arange(E_PAD - E, dtype=jnp.int32) % (N_PAD - N))
    src_ob0 = jnp.concatenate([src, pad_oob0])
    dst_ob0 = jnp.concatenate([dst, pad_oob0])
    mem_pad = jnp.pad(memory, ((0, N_PAD - N), (0, 0)))
    zrows = jnp.zeros((656, D), jnp.float32)
    S_pad = _scatter_add(src_ob0, dst_ob0, mem_pad, zrows)

    H, B2 = _dense_phase(memory, S_pad, Wcat, W_nbr, b[None, :])

    pad_ids = (jnp.arange(E_PAD - E, dtype=jnp.int32) * 7919) % N
    src_p = jnp.concatenate([src, pad_ids])
    dst_p = jnp.concatenate([dst, pad_ids])
    idx_all = jnp.concatenate([src_p, dst_p])
    hsd = _gather_rows(H, idx_all, D)
    h_src = hsd[:E]
    h_dst = hsd[E_PAD:E_PAD + E]

    pad_oob = N + (jnp.arange(E_PAD - E, dtype=jnp.int32) % (N_PAD - N))
    src_ob = jnp.concatenate([src, pad_oob])
    dst_ob = jnp.concatenate([dst, pad_oob])
    p2 = _winner(src_ob, dst_ob)
    p1 = jnp.maximum(p2[0, :N], p2[1, :N])

    valid = (p1 > 0).astype(jnp.int32)
    is_dst = p1 > E_PAD
    ew = jnp.where(is_dst, p1 - 1 - E_PAD, p1 - 1)
    spread = jnp.arange(N, dtype=jnp.int32) % E
    ew_c = jnp.where(p1 > 0, jnp.clip(ew, 0, E - 1), spread)
    opp = jnp.where(is_dst, src[ew_c], dst[ew_c])
    dtw = t[ew_c] - last_update
    new_lu = jnp.where(valid > 0, t[ew_c], last_update)
    msg_g = msg[ew_c]
    B2g = B2[opp]

    new_memory = _final_phase(W_upd[:D], B2g, msg_g, dtw[:, None], valid[:, None],
                              memory, Wu3, Wu4, b_upd[None, :], w_time[None, :])

    return (h_src, h_dst, new_memory, new_lu)


# speedup vs baseline: 4.0114x; 1.1072x over previous
import functools

import jax
import jax.numpy as jnp
from jax import lax
from jax.experimental import pallas as pl
from jax.experimental.pallas import tpu as pltpu
from jax.experimental.pallas import tpu_sc as plsc

N = 100000
D = 128
E = 100000
MSG = 16
TD = 16

BR = 2000

NC = 2
NS = 16
NW = NC * NS
E_PAD = 102400

_SC_MESH = plsc.VectorSubcoreMesh(core_axis_name="c", subcore_axis_name="s")


def _gather_rows(table, idx, d, kb=320):
    m = idx.shape[0]
    pw = m // NW
    nb = pw // kb
    assert m % (NW * kb) == 0 and nb % 2 == 0

    @functools.partial(
        pl.kernel,
        out_type=jax.ShapeDtypeStruct((m, d), jnp.float32),
        mesh=_SC_MESH,
        scratch_types=[
            pltpu.VMEM((pw,), jnp.int32),
            pltpu.VMEM((kb, d), jnp.float32),
            pltpu.VMEM((kb, d), jnp.float32),
            pltpu.SemaphoreType.DMA,
            pltpu.SemaphoreType.DMA,
        ],
    )
    def gk(idx_hbm, tbl_hbm, out_hbm, idx_v, buf0, buf1, sem0, sem1):
        wid = lax.axis_index("s") * NC + lax.axis_index("c")
        base = wid * pw
        pltpu.sync_copy(idx_hbm.at[pl.ds(base, pw)], idx_v)
        pltpu.async_copy(tbl_hbm.at[idx_v.at[pl.ds(0, kb)]], buf0, sem0)

        @pl.loop(0, nb, step=2)
        def _(b):
            pltpu.make_async_copy(tbl_hbm.at[pl.ds(0, kb)], buf0, sem0).wait()
            pltpu.async_copy(tbl_hbm.at[idx_v.at[pl.ds((b + 1) * kb, kb)]],
                             buf1, sem1)
            pltpu.sync_copy(buf0, out_hbm.at[pl.ds(base + b * kb, kb)])

            @pl.when(b + 2 < nb)
            def _():
                pltpu.async_copy(tbl_hbm.at[idx_v.at[pl.ds((b + 2) * kb, kb)]],
                                 buf0, sem0)

            pltpu.make_async_copy(tbl_hbm.at[pl.ds(0, kb)], buf1, sem1).wait()
            pltpu.sync_copy(buf1, out_hbm.at[pl.ds(base + (b + 1) * kb, kb)])

    return gk(idx, table)


def _gather_rows_split(table, idx, d, kb=80):
    m = idx.shape[0]
    pw = m // NW
    nb = pw // kb
    ndump = m - 2 * E
    assert m % (NW * kb) == 0 and nb % 2 == 0 and E % kb == 0

    @functools.partial(
        pl.kernel,
        out_type=[jax.ShapeDtypeStruct((E, d), jnp.float32),
                  jax.ShapeDtypeStruct((E, d), jnp.float32),
                  jax.ShapeDtypeStruct((ndump, d), jnp.float32)],
        mesh=_SC_MESH,
        scratch_types=[
            pltpu.VMEM((pw,), jnp.int32),
            pltpu.VMEM((kb, d), jnp.float32),
            pltpu.VMEM((kb, d), jnp.float32),
            pltpu.SemaphoreType.DMA,
            pltpu.SemaphoreType.DMA,
        ],
    )
    def gk(idx_hbm, tbl_hbm, o0_hbm, o1_hbm, od_hbm, idx_v, buf0, buf1,
           sem0, sem1):
        wid = lax.axis_index("s") * NC + lax.axis_index("c")
        base = wid * pw
        pltpu.sync_copy(idx_hbm.at[pl.ds(base, pw)], idx_v)
        pltpu.async_copy(tbl_hbm.at[idx_v.at[pl.ds(0, kb)]], buf0, sem0)

        def write(buf, b):
            flat = base + b * kb

            @pl.when(flat < E)
            def _():
                pltpu.sync_copy(buf, o0_hbm.at[pl.ds(flat, kb)])

            @pl.when((flat >= E) & (flat < 2 * E))
            def _():
                pltpu.sync_copy(buf, o1_hbm.at[pl.ds(flat - E, kb)])

            @pl.when(flat >= 2 * E)
            def _():
                pltpu.sync_copy(buf, od_hbm.at[pl.ds(flat - 2 * E, kb)])

        @pl.loop(0, nb, step=2)
        def _(b):
            pltpu.make_async_copy(tbl_hbm.at[pl.ds(0, kb)], buf0, sem0).wait()
            pltpu.async_copy(tbl_hbm.at[idx_v.at[pl.ds((b + 1) * kb, kb)]],
                             buf1, sem1)
            write(buf0, b)

            @pl.when(b + 2 < nb)
            def _():
                pltpu.async_copy(tbl_hbm.at[idx_v.at[pl.ds((b + 2) * kb, kb)]],
                                 buf0, sem0)

            pltpu.make_async_copy(tbl_hbm.at[pl.ds(0, kb)], buf1, sem1).wait()
            write(buf1, b + 1)

    return gk(idx, table)


N_PAD = 102400


def _winner(src_p, dst_p):
    epw = E_PAD // NW
    ngrp = epw // 16
    CHK = 10240
    NR = N_PAD // CHK
    mpt = CHK // NS

    @functools.partial(
        pl.kernel,
        out_type=jax.ShapeDtypeStruct((2, N_PAD), jnp.int32),
        mesh=_SC_MESH,
        compiler_params=pltpu.CompilerParams(needs_layout_passes=False),
        scratch_types=[
            pltpu.VMEM((epw,), jnp.int32),
            pltpu.VMEM((epw,), jnp.int32),
            pltpu.VMEM((N_PAD,), jnp.int32),
            pltpu.VMEM((NS, mpt), jnp.int32),
            pltpu.VMEM((mpt,), jnp.int32),
            pltpu.VMEM_SHARED((NS, CHK), jnp.int32),
        ],
    )
    def wk(src_hbm, dst_hbm, out_hbm, sbuf, dbuf, win, mbuf, tmp, shared):
        cid = lax.axis_index("c")
        sid = lax.axis_index("s")
        wid = sid * NC + cid
        ebase = wid * epw
        pltpu.sync_copy(src_hbm.at[pl.ds(ebase, epw)], sbuf)
        pltpu.sync_copy(dst_hbm.at[pl.ds(ebase, epw)], dbuf)

        @pl.loop(0, N_PAD // 16)
        def _(i):
            win[pl.ds(i * 16, 16)] = jnp.zeros((16,), jnp.int32)

        lanes = lax.iota(jnp.int32, 16)

        @pl.loop(0, ngrp)
        def _(g):
            pri0 = ebase + g * 16 + lanes + 1
            for buf, off in ((sbuf, 0), (dbuf, E_PAD)):
                ids = buf[pl.ds(g * 16, 16)]
                pri = pri0 + off

                def attempt(_):
                    cur = plsc.load_gather(win, [ids])
                    plsc.store_scatter(win, [ids], jnp.maximum(cur, pri))
                    cur2 = plsc.load_gather(win, [ids])
                    return jnp.any(cur2 < pri)

                lax.while_loop(lambda need: need, attempt, attempt(None))

        for k in range(NR):
            pltpu.sync_copy(win.at[pl.ds(k * CHK, CHK)], shared.at[sid])
            plsc.subcore_barrier()
            pltpu.sync_copy(shared.at[:, pl.ds(sid * mpt, mpt)], mbuf)

            @pl.loop(0, mpt // 16)
            def _(g):
                sl = pl.ds(g * 16, 16)
                acc = mbuf[0, sl]
                for r in range(1, NS):
                    acc = jnp.maximum(acc, mbuf[r, sl])
                tmp[sl] = acc

            pltpu.sync_copy(tmp, out_hbm.at[cid, pl.ds(k * CHK + sid * mpt, mpt)])
            plsc.subcore_barrier()

    return wk(src_p, dst_p)


def _scatter_add(src_ob, dst_ob, mem_pad, zrows):
    CH = 10240
    NCHUNK = N_PAD // CH
    ZR = CH // NS + 16
    SROWS = NS * ZR
    epc = E_PAD // NS
    GB = 128
    CAP = epc + 2 * GB

    @functools.partial(
        pl.kernel,
        out_type=jax.ShapeDtypeStruct((N_PAD, D), jnp.float32),
        mesh=_SC_MESH,
        compiler_params=pltpu.CompilerParams(needs_layout_passes=False),
        scratch_types=[
            pltpu.VMEM((CAP,), jnp.int32),
            pltpu.VMEM((CAP,), jnp.int32),
            pltpu.VMEM((GB, D), jnp.float32),
            pltpu.VMEM((GB, D), jnp.float32),
            pltpu.VMEM((GB,), jnp.int32),
            pltpu.VMEM((GB,), jnp.int32),
            pltpu.VMEM((GB,), jnp.int32),
            pltpu.VMEM((GB,), jnp.int32),
            pltpu.VMEM_SHARED((SROWS, D), jnp.float32),
            pltpu.SemaphoreType.DMA,
            pltpu.SemaphoreType.DMA,
        ],
    )
    def sk(src_hbm, dst_hbm, mem_hbm, z_hbm, out_hbm,
           sbuf, dbuf, rows0, rows1, idxg0, idxg1, lidg0, lidg1,
           sloc, semg0, semg1):
        cid = lax.axis_index("c")
        sid = lax.axis_index("s")
        lanes = lax.iota(jnp.int32, 16)

        for kc in range(NCHUNK // NC):
            lo = (NC * kc + cid) * CH
            pltpu.sync_copy(z_hbm, sloc.at[pl.ds(sid * ZR, ZR)])
            pltpu.sync_copy(src_hbm.at[pl.ds(sid * epc, epc)], sbuf.at[pl.ds(0, epc)])
            pltpu.sync_copy(dst_hbm.at[pl.ds(sid * epc, epc)], dbuf.at[pl.ds(0, epc)])
            plsc.subcore_barrier()

            @pl.loop(0, epc // 16, init_carry=jnp.int32(0))
            def compact(g, off):
                ids_d = dbuf[pl.ds(g * 16, 16)]
                ids_s = sbuf[pl.ds(g * 16, 16)]
                lid = ids_d - lo
                mask = (lid >= 0) & (lid < CH)
                plsc.store_compressed(dbuf.at[pl.ds(off, 16)], lid, mask=mask)
                plsc.store_compressed(sbuf.at[pl.ds(off, 16)], ids_s, mask=mask)
                return off + jnp.sum(mask.astype(jnp.int32))

            off = compact
            for j in range(2 * GB // 16):
                dbuf[pl.ds(off + j * 16, 16)] = CH + lanes
                sbuf[pl.ds(off + j * 16, 16)] = lanes

            npair = jnp.maximum((off + 2 * GB - 1) // (2 * GB), 1)

            def stage(b, idxg, lidg):
                for k in range(GB // 16):
                    sl16 = pl.ds(k * 16, 16)
                    idxg[sl16] = sbuf[pl.ds(b * GB + k * 16, 16)]
                    lidg[sl16] = dbuf[pl.ds(b * GB + k * 16, 16)]

            stage(jnp.int32(0), idxg0, lidg0)
            pltpu.async_copy(mem_hbm.at[idxg0], rows0, semg0)

            @pl.loop(0, npair)
            def gather_scatter(i):
                b0 = 2 * i
                pltpu.make_async_copy(mem_hbm.at[pl.ds(0, GB)], rows0, semg0).wait()
                stage(b0 + 1, idxg1, lidg1)
                pltpu.async_copy(mem_hbm.at[idxg1], rows1, semg1)
                pltpu.sync_copy(rows0, sloc.at[lidg0], add=True)

                @pl.when(i + 1 < npair)
                def _():
                    stage(b0 + 2, idxg0, lidg0)
                    pltpu.async_copy(mem_hbm.at[idxg0], rows0, semg0)

                pltpu.make_async_copy(mem_hbm.at[pl.ds(0, GB)], rows1, semg1).wait()
                pltpu.sync_copy(rows1, sloc.at[lidg1], add=True)

            plsc.subcore_barrier()
            wb = CH // NS
            pltpu.sync_copy(sloc.at[pl.ds(sid * wb, wb)],
                            out_hbm.at[pl.ds(lo + sid * wb, wb)])
            plsc.subcore_barrier()

    return sk(src_ob, dst_ob, mem_pad, zrows)


def _dense_body(mem_ref, s_ref, wcat_ref, wnbr_ref, b_ref, h_ref, b2_ref):
    mem = mem_ref[...]
    c2 = jnp.dot(mem, wcat_ref[...], preferred_element_type=jnp.float32)
    hpre = c2[:, :D] + jnp.dot(s_ref[...], wnbr_ref[...],
                               preferred_element_type=jnp.float32) + b_ref[...]
    h_ref[...] = jnp.tanh(hpre)
    b2_ref[...] = c2[:, D:2 * D]


def _dense_phase(memory, S, Wcat, W_nbr, b):
    grid = (N // BR,)
    return pl.pallas_call(
        _dense_body,
        grid=grid,
        in_specs=[
            pl.BlockSpec((BR, D), lambda i: (i, 0)),
            pl.BlockSpec((BR, D), lambda i: (i, 0)),
            pl.BlockSpec((D, 2 * D), lambda i: (0, 0)),
            pl.BlockSpec((D, D), lambda i: (0, 0)),
            pl.BlockSpec((1, D), lambda i: (0, 0)),
        ],
        out_specs=[
            pl.BlockSpec((BR, D), lambda i: (i, 0)),
            pl.BlockSpec((BR, D), lambda i: (i, 0)),
        ],
        out_shape=[jax.ShapeDtypeStruct((N, D), jnp.float32)] * 2,
    )(memory, S, Wcat, W_nbr, b)


def _final_body(wu1_ref, b2g_ref, msgg_ref, dtw_ref, valid_ref, mem_ref,
                wu3_ref, wu4_ref, bupd_ref, wt_ref, out_ref):
    te = jnp.cos(dtw_ref[...] * wt_ref[...])
    a_blk = jnp.dot(mem_ref[...], wu1_ref[...], preferred_element_type=jnp.float32)
    pre = (a_blk + b2g_ref[...]
           + jnp.dot(msgg_ref[...], wu3_ref[...], preferred_element_type=jnp.float32)
           + jnp.dot(te, wu4_ref[...], preferred_element_type=jnp.float32)
           + bupd_ref[...])
    m = jnp.tanh(pre)
    out_ref[...] = jnp.where(valid_ref[...] > 0, m, mem_ref[...])


def _final_phase(Wu1, B2g, msg_g, dtw, valid, memory, Wu3, Wu4, b_upd, w_time):
    grid = (N // BR,)
    return pl.pallas_call(
        _final_body,
        grid=grid,
        in_specs=[
            pl.BlockSpec((D, D), lambda i: (0, 0)),
            pl.BlockSpec((BR, D), lambda i: (i, 0)),
            pl.BlockSpec((BR, MSG), lambda i: (i, 0)),
            pl.BlockSpec((BR, 1), lambda i: (i, 0)),
            pl.BlockSpec((BR, 1), lambda i: (i, 0)),
            pl.BlockSpec((BR, D), lambda i: (i, 0)),
            pl.BlockSpec((MSG, D), lambda i: (0, 0)),
            pl.BlockSpec((TD, D), lambda i: (0, 0)),
            pl.BlockSpec((1, D), lambda i: (0, 0)),
            pl.BlockSpec((1, TD), lambda i: (0, 0)),
        ],
        out_specs=pl.BlockSpec((BR, D), lambda i: (i, 0)),
        out_shape=jax.ShapeDtypeStruct((N, D), jnp.float32),
    )(Wu1, B2g, msg_g, dtw, valid, memory, Wu3, Wu4, b_upd, w_time)


def kernel(edge_index, t, msg, memory, last_update, W_self, W_nbr, b, W_upd, b_upd, w_time):
    src, dst = edge_index[0], edge_index[1]
    Wcat = jnp.concatenate([W_self, W_upd[D:2 * D]], axis=1)
    Wu3 = W_upd[2 * D:2 * D + MSG]
    Wu4 = W_upd[2 * D + MSG:]

    pad_oob0 = N + (jnp.arange(E_PAD - E, dtype=jnp.int32) % (N_PAD - N))
    pad_safe = (jnp.arange(E_PAD - E, dtype=jnp.int32) * 7919) % N
    src_ob0 = jnp.concatenate([src, pad_safe])
    dst_ob0 = jnp.concatenate([dst, pad_oob0])
    zrows = jnp.zeros((656, D), jnp.float32)
    S_pad = _scatter_add(src_ob0, dst_ob0, memory, zrows)

    H, B2 = _dense_phase(memory, S_pad, Wcat, W_nbr, b[None, :])

    idx_all = jnp.concatenate([src, dst, pad_safe, pad_safe])
    h_src, h_dst, _ = _gather_rows_split(H, idx_all, D)

    pad_oob = N + (jnp.arange(E_PAD - E, dtype=jnp.int32) % (N_PAD - N))
    src_ob = jnp.concatenate([src, pad_oob])
    dst_ob = jnp.concatenate([dst, pad_oob])
    p2 = _winner(src_ob, dst_ob)
    p1 = jnp.maximum(p2[0, :N], p2[1, :N])

    valid = (p1 > 0).astype(jnp.int32)
    is_dst = p1 > E_PAD
    ew = jnp.where(is_dst, p1 - 1 - E_PAD, p1 - 1)
    spread = jnp.arange(N, dtype=jnp.int32) % E
    ew_c = jnp.where(p1 > 0, jnp.clip(ew, 0, E - 1), spread)
    opp = jnp.where(is_dst, src[ew_c], dst[ew_c])
    dtw = t[ew_c] - last_update
    new_lu = jnp.where(valid > 0, t[ew_c], last_update)
    msg_g = msg[ew_c]
    B2g = B2[opp]

    new_memory = _final_phase(W_upd[:D], B2g, msg_g, dtw[:, None], valid[:, None],
                              memory, Wu3, Wu4, b_upd[None, :], w_time[None, :])

    return (h_src, h_dst, new_memory, new_lu)
